# initial kernel scaffold (unmeasured)
import jax
import jax.numpy as jnp
from jax import lax
from jax.experimental import pallas as pl
from jax.experimental.pallas import tpu as pltpu

N_GLOBAL_COLS = 4096
EPS = 1e-5
TM = 512


def kernel(x, gamma):
    m, n = x.shape
    nblk = m // TM
    gamma2 = gamma.reshape(1, n)

    def body(x_ref, g_ref, out_ref, ot, partial, recv, out_sems, send_sem, recv_sem):
        my_x = lax.axis_index("x")
        my_y = lax.axis_index("y")
        nbr = (my_x, 1 - my_y)

        barrier = pltpu.get_barrier_semaphore()
        pl.semaphore_signal(
            barrier, inc=1, device_id=nbr, device_id_type=pl.DeviceIdType.MESH
        )
        pl.semaphore_wait(barrier, 1)

        for i in range(nblk):
            blk = x_ref[pl.ds(i * TM, TM), :]
            partial[pl.ds(i * TM, TM), :] = jnp.sum(blk * blk, axis=1, keepdims=True)

        rdma = pltpu.make_async_remote_copy(
            src_ref=partial,
            dst_ref=recv,
            send_sem=send_sem,
            recv_sem=recv_sem,
            device_id=nbr,
            device_id_type=pl.DeviceIdType.MESH,
        )
        rdma.start()
        rdma.wait()

        total = partial[:, :] + recv[:, :]
        partial[:, :] = lax.rsqrt(total * (1.0 / N_GLOBAL_COLS) + EPS)

        copies = [None] * nblk
        for i in range(nblk):
            slot = i % 2
            if i >= 2:
                copies[i - 2].wait()
            ot[slot, :, :] = (
                x_ref[pl.ds(i * TM, TM), :]
                * g_ref[:, :]
                * partial[pl.ds(i * TM, TM), :]
            )
            copies[i] = pltpu.make_async_copy(
                ot.at[slot], out_ref.at[pl.ds(i * TM, TM), :], out_sems.at[slot]
            )
            copies[i].start()
        for i in range(max(nblk - 2, 0), nblk):
            copies[i].wait()

    return pl.pallas_call(
        body,
        out_shape=jax.ShapeDtypeStruct((m, n), x.dtype),
        in_specs=[
            pl.BlockSpec(memory_space=pltpu.VMEM),
            pl.BlockSpec(memory_space=pltpu.VMEM),
        ],
        out_specs=pl.BlockSpec(memory_space=pltpu.ANY),
        scratch_shapes=[
            pltpu.VMEM((2, TM, n), jnp.float32),
            pltpu.VMEM((m, 1), jnp.float32),
            pltpu.VMEM((m, 1), jnp.float32),
            pltpu.SemaphoreType.DMA((2,)),
            pltpu.SemaphoreType.DMA,
            pltpu.SemaphoreType.DMA,
        ],
        compiler_params=pltpu.CompilerParams(collective_id=0),
    )(x, gamma2)


# baseline (device time: 157930 ns/iter reference)
import jax
import jax.numpy as jnp
from jax import lax
from jax.experimental import pallas as pl
from jax.experimental.pallas import tpu as pltpu

N_GLOBAL_COLS = 4096
EPS = 1e-5
TM = 512


def kernel(x, gamma):
    m, n = x.shape
    nblk = m // TM
    gamma2 = gamma.reshape(1, n)

    def body(
        x_hbm, g_ref, out_hbm,
        xt, ot, partial, recv,
        in_sems, out_sems, send_sem, recv_sem,
    ):
        my_x = lax.axis_index("x")
        my_y = lax.axis_index("y")
        nbr = (my_x, 1 - my_y)

        barrier = pltpu.get_barrier_semaphore()
        pl.semaphore_signal(
            barrier, inc=1, device_id=nbr, device_id_type=pl.DeviceIdType.MESH
        )
        pl.semaphore_wait(barrier, 1)

        def in_copy(i):
            return pltpu.make_async_copy(
                x_hbm.at[pl.ds(i * TM, TM), :], xt.at[i % 2], in_sems.at[i % 2]
            )

        in_copy(0).start()
        for i in range(nblk):
            if i + 1 < nblk:
                in_copy(i + 1).start()
            in_copy(i).wait()
            blk = xt[i % 2]
            partial[pl.ds(i * TM, TM), :] = jnp.sum(blk * blk, axis=1, keepdims=True)

        rdma = pltpu.make_async_remote_copy(
            src_ref=partial,
            dst_ref=recv,
            send_sem=send_sem,
            recv_sem=recv_sem,
            device_id=nbr,
            device_id_type=pl.DeviceIdType.MESH,
        )
        rdma.start()
        rdma.wait()

        total = partial[:, :] + recv[:, :]
        partial[:, :] = lax.rsqrt(total * (1.0 / N_GLOBAL_COLS) + EPS)

        out_copies = [None] * nblk
        in_copy(0).start()
        for i in range(nblk):
            if i + 1 < nblk:
                in_copy(i + 1).start()
            in_copy(i).wait()
            if i >= 2:
                out_copies[i - 2].wait()
            ot[i % 2, :, :] = (
                xt[i % 2] * g_ref[:, :] * partial[pl.ds(i * TM, TM), :]
            )
            out_copies[i] = pltpu.make_async_copy(
                ot.at[i % 2], out_hbm.at[pl.ds(i * TM, TM), :], out_sems.at[i % 2]
            )
            out_copies[i].start()
        for i in range(max(nblk - 2, 0), nblk):
            out_copies[i].wait()

    return pl.pallas_call(
        body,
        out_shape=jax.ShapeDtypeStruct((m, n), x.dtype),
        in_specs=[
            pl.BlockSpec(memory_space=pl.ANY),
            pl.BlockSpec(memory_space=pltpu.VMEM),
        ],
        out_specs=pl.BlockSpec(memory_space=pl.ANY),
        scratch_shapes=[
            pltpu.VMEM((2, TM, n), jnp.float32),
            pltpu.VMEM((2, TM, n), jnp.float32),
            pltpu.VMEM((m, 1), jnp.float32),
            pltpu.VMEM((m, 1), jnp.float32),
            pltpu.SemaphoreType.DMA((2,)),
            pltpu.SemaphoreType.DMA((2,)),
            pltpu.SemaphoreType.DMA,
            pltpu.SemaphoreType.DMA,
        ],
        compiler_params=pltpu.CompilerParams(collective_id=0),
    )(x, gamma2)
